# Initial kernel scaffold; baseline (speedup 1.0000x reference)
import jax
import jax.numpy as jnp
from jax.experimental import pallas as pl


def kernel(x, edge_index, batch, W1, b1, g1, be1, W2, b2, g2, be2, W3, b3, g3, be3, Wf, bf):
    raise NotImplementedError("write your pallas kernel here")


# speedup vs baseline: 19.9103x; 19.9103x over previous
import functools

import jax
import jax.numpy as jnp
from jax import lax
from jax.experimental import pallas as pl
from jax.experimental.pallas import tpu as pltpu
from jax.experimental.pallas import tpu_sc as plsc

N = 100000
G = 512
BLK = 2048
GRID = 49
NP = BLK * GRID
E = 1600000
NW = 32
ROWS_W = 392
EP = NW * ROWS_W * 128
CHUNKS = 49
RPS = NP // 16


@functools.cache
def _mesh():
    return plsc.VectorSubcoreMesh(
        core_axis_name="c", subcore_axis_name="s",
        num_cores=2, num_subcores=16)


@functools.cache
def _make_deg():
    @functools.partial(
        pl.kernel,
        out_type=jax.ShapeDtypeStruct((2, NP, 16), jnp.float32),
        mesh=_mesh(),
        compiler_params=pltpu.CompilerParams(use_tc_tiling_on_sc=False),
        scratch_types=[
            pltpu.VMEM_SHARED((NP, 16), jnp.float32),
            pltpu.VMEM((8, 128), jnp.int32),
            pltpu.VMEM((128, 16), jnp.float32),
        ],
    )
    def _deg_kernel(dst_h, zeros_h, ones_h, out_h, acc, dstb, onesb):
        c = lax.axis_index("c")
        s = lax.axis_index("s")
        w = s * 2 + c
        pltpu.sync_copy(ones_h, onesb)
        pltpu.sync_copy(zeros_h.at[pl.ds(s * RPS, RPS)],
                        acc.at[pl.ds(s * RPS, RPS)])
        plsc.subcore_barrier()

        def chunk(i, carry):
            r0 = w * ROWS_W + i * 8
            pltpu.sync_copy(dst_h.at[pl.ds(r0, 8)], dstb)
            for k in range(8):
                pltpu.sync_copy(onesb, acc.at[dstb.at[k]], add=True)
            return carry

        lax.fori_loop(0, CHUNKS, chunk, 0)
        plsc.subcore_barrier()
        pltpu.sync_copy(acc.at[pl.ds(s * RPS, RPS)],
                        out_h.at[c, pl.ds(s * RPS, RPS)])

    return _deg_kernel


@functools.cache
def _make_agg(F):
    @functools.partial(
        pl.kernel,
        out_type=jax.ShapeDtypeStruct((2, NP, F), jnp.float32),
        mesh=_mesh(),
        compiler_params=pltpu.CompilerParams(use_tc_tiling_on_sc=False),
        scratch_types=[
            pltpu.VMEM_SHARED((NP, F), jnp.float32),
            pltpu.VMEM((8, 128), jnp.int32),
            pltpu.VMEM((8, 128), jnp.int32),
            pltpu.VMEM((8, 128, F), jnp.float32),
            pltpu.SemaphoreType.DMA,
            pltpu.SemaphoreType.DMA,
        ],
    )
    def _agg(table_h, src_h, dst_h, zeros_h, out_h, acc, srcb, dstb, rows,
             gsem, ssem):
        c = lax.axis_index("c")
        s = lax.axis_index("s")
        w = s * 2 + c
        pltpu.sync_copy(zeros_h.at[pl.ds(s * RPS, RPS)],
                        acc.at[pl.ds(s * RPS, RPS)])
        plsc.subcore_barrier()

        def chunk(i, carry):
            r0 = w * ROWS_W + i * 8
            pltpu.sync_copy(src_h.at[pl.ds(r0, 8)], srcb)
            pltpu.sync_copy(dst_h.at[pl.ds(r0, 8)], dstb)
            gets = [pltpu.async_copy(table_h.at[srcb.at[k]], rows.at[k], gsem)
                    for k in range(8)]
            for cp in gets:
                cp.wait()
            puts = [pltpu.async_copy(rows.at[k], acc.at[dstb.at[k]], ssem,
                                     add=True)
                    for k in range(8)]
            for cp in puts:
                cp.wait()
            return carry

        lax.fori_loop(0, CHUNKS, chunk, 0)
        plsc.subcore_barrier()
        pltpu.sync_copy(acc.at[pl.ds(s * RPS, RPS)],
                        out_h.at[c, pl.ds(s * RPS, RPS)])

    return _agg


def _sc_deg(dst2d, zeros16, ones128):
    return _make_deg()(dst2d, zeros16, ones128)


def _sc_agg(table, src2d, dst2d, zerosF):
    return _make_agg(table.shape[1])(table, src2d, dst2d, zerosF)


def _dot(a, b):
    return lax.dot_general(a, b, (((1,), (0,)), ((), ())),
                           precision=lax.Precision.HIGHEST,
                           preferred_element_type=jnp.float32)


def _prep_body(d0_ref, d1_ref, valid_ref, x16_ref, dinv_ref, t1_ref):
    d = d0_ref[:, 0:1] + d1_ref[:, 0:1] + 1.0
    dinv = valid_ref[...] * lax.rsqrt(d)
    dinv_ref[...] = dinv
    t1_ref[...] = dinv * x16_ref[...]


def _tc_prep(d0, d1, valid, x16):
    return pl.pallas_call(
        _prep_body,
        grid=(GRID,),
        in_specs=[
            pl.BlockSpec((BLK, 16), lambda i: (i, 0)),
            pl.BlockSpec((BLK, 16), lambda i: (i, 0)),
            pl.BlockSpec((BLK, 1), lambda i: (i, 0)),
            pl.BlockSpec((BLK, 16), lambda i: (i, 0)),
        ],
        out_specs=[
            pl.BlockSpec((BLK, 1), lambda i: (i, 0)),
            pl.BlockSpec((BLK, 16), lambda i: (i, 0)),
        ],
        out_shape=[
            jax.ShapeDtypeStruct((NP, 1), jnp.float32),
            jax.ShapeDtypeStruct((NP, 16), jnp.float32),
        ],
    )(d0, d1, valid, x16)


def _stats_body(s0_ref, s1_ref, t_ref, dinv_ref, z_ref, m_ref):
    i = pl.program_id(0)
    z = dinv_ref[...] * (s0_ref[...] + s1_ref[...] + t_ref[...])
    z_ref[...] = z
    ze = jnp.concatenate([z, jnp.ones((BLK, 1), jnp.float32)], axis=1)
    m = lax.dot_general(ze, ze, (((0,), (0,)), ((), ())),
                        precision=lax.Precision.HIGHEST,
                        preferred_element_type=jnp.float32)

    @pl.when(i == 0)
    def _():
        m_ref[...] = m

    @pl.when(i > 0)
    def _():
        m_ref[...] += m


def _tc_stats(s0, s1, t, dinv):
    F = t.shape[1]
    return pl.pallas_call(
        _stats_body,
        grid=(GRID,),
        in_specs=[
            pl.BlockSpec((BLK, F), lambda i: (i, 0)),
            pl.BlockSpec((BLK, F), lambda i: (i, 0)),
            pl.BlockSpec((BLK, F), lambda i: (i, 0)),
            pl.BlockSpec((BLK, 1), lambda i: (i, 0)),
        ],
        out_specs=[
            pl.BlockSpec((BLK, F), lambda i: (i, 0)),
            pl.BlockSpec((F + 1, F + 1), lambda i: (0, 0)),
        ],
        out_shape=[
            jax.ShapeDtypeStruct((NP, F), jnp.float32),
            jax.ShapeDtypeStruct((F + 1, F + 1), jnp.float32),
        ],
    )(s0, s1, t, dinv)


def _stats2_body(s0a_ref, s1a_ref, ta_ref, s0b_ref, s1b_ref, tb_ref,
                 dinv_ref, z_ref, m_ref):
    i = pl.program_id(0)
    dinv = dinv_ref[...]
    za = dinv * (s0a_ref[...] + s1a_ref[...] + ta_ref[...])
    zb = dinv * (s0b_ref[...] + s1b_ref[...] + tb_ref[...])
    z = jnp.concatenate([za, zb], axis=1)
    z_ref[...] = z
    ze = jnp.concatenate([z, jnp.ones((BLK, 1), jnp.float32)], axis=1)
    m = lax.dot_general(ze, ze, (((0,), (0,)), ((), ())),
                        precision=lax.Precision.HIGHEST,
                        preferred_element_type=jnp.float32)

    @pl.when(i == 0)
    def _():
        m_ref[...] = m

    @pl.when(i > 0)
    def _():
        m_ref[...] += m


def _tc_stats2(s0a, s1a, ta, s0b, s1b, tb, dinv):
    spec16 = pl.BlockSpec((BLK, 16), lambda i: (i, 0))
    return pl.pallas_call(
        _stats2_body,
        grid=(GRID,),
        in_specs=[spec16, spec16, spec16, spec16, spec16, spec16,
                  pl.BlockSpec((BLK, 1), lambda i: (i, 0))],
        out_specs=[
            pl.BlockSpec((BLK, 32), lambda i: (i, 0)),
            pl.BlockSpec((33, 33), lambda i: (0, 0)),
        ],
        out_shape=[
            jax.ShapeDtypeStruct((NP, 32), jnp.float32),
            jax.ShapeDtypeStruct((33, 33), jnp.float32),
        ],
    )(s0a, s1a, ta, s0b, s1b, tb, dinv)


def _finalize_bn(M, W):
    F = W.shape[0]
    n = jnp.float32(N)
    mu_z = M[:F, F] / n
    cov = M[:F, :F] / n - jnp.outer(mu_z, mu_z)
    mu_c = (mu_z @ W).reshape(1, -1)
    var_c = jnp.einsum('if,ij,jf->f', W, cov, W)
    istd = lax.rsqrt(jnp.maximum(var_c, 0.0) + 1e-5).reshape(1, -1)
    return mu_c, istd


def _apply_body(z_ref, dinv_ref, w_ref, mu_ref, istd_ref, g_ref, be_ref,
                t_ref):
    c = _dot(z_ref[...], w_ref[...])
    y = jnp.maximum((c - mu_ref[...]) * istd_ref[...] * g_ref[...]
                    + be_ref[...], 0.0)
    t_ref[...] = dinv_ref[...] * y


def _tc_apply1(z, dinv, W, mu, istd, g, be):
    return pl.pallas_call(
        _apply_body,
        grid=(GRID,),
        in_specs=[
            pl.BlockSpec((BLK, 16), lambda i: (i, 0)),
            pl.BlockSpec((BLK, 1), lambda i: (i, 0)),
            pl.BlockSpec((16, 16), lambda i: (0, 0)),
            pl.BlockSpec((1, 16), lambda i: (0, 0)),
            pl.BlockSpec((1, 16), lambda i: (0, 0)),
            pl.BlockSpec((1, 16), lambda i: (0, 0)),
            pl.BlockSpec((1, 16), lambda i: (0, 0)),
        ],
        out_specs=pl.BlockSpec((BLK, 16), lambda i: (i, 0)),
        out_shape=jax.ShapeDtypeStruct((NP, 16), jnp.float32),
    )(z, dinv, W, mu, istd, g, be)


def _apply2_body(z_ref, dinv_ref, w_ref, mu_ref, istd_ref, g_ref, be_ref,
                 ta_ref, tb_ref):
    c = _dot(z_ref[...], w_ref[...])
    y = jnp.maximum((c - mu_ref[...]) * istd_ref[...] * g_ref[...]
                    + be_ref[...], 0.0)
    t = dinv_ref[...] * y
    ta_ref[...] = t[:, :16]
    tb_ref[...] = t[:, 16:]


def _tc_apply2(z, dinv, W, mu, istd, g, be):
    return pl.pallas_call(
        _apply2_body,
        grid=(GRID,),
        in_specs=[
            pl.BlockSpec((BLK, 16), lambda i: (i, 0)),
            pl.BlockSpec((BLK, 1), lambda i: (i, 0)),
            pl.BlockSpec((16, 32), lambda i: (0, 0)),
            pl.BlockSpec((1, 32), lambda i: (0, 0)),
            pl.BlockSpec((1, 32), lambda i: (0, 0)),
            pl.BlockSpec((1, 32), lambda i: (0, 0)),
            pl.BlockSpec((1, 32), lambda i: (0, 0)),
        ],
        out_specs=[
            pl.BlockSpec((BLK, 16), lambda i: (i, 0)),
            pl.BlockSpec((BLK, 16), lambda i: (i, 0)),
        ],
        out_shape=[
            jax.ShapeDtypeStruct((NP, 16), jnp.float32),
            jax.ShapeDtypeStruct((NP, 16), jnp.float32),
        ],
    )(z, dinv, W, mu, istd, g, be)


def _pool_body(z_ref, b_ref, w_ref, mu_ref, istd_ref, g_ref, be_ref,
               wf_ref, bf_ref, out_ref, acc_ref):
    i = pl.program_id(0)
    c = _dot(z_ref[...], w_ref[...])
    y = jnp.maximum((c - mu_ref[...]) * istd_ref[...] * g_ref[...]
                    + be_ref[...], 0.0)
    ye = jnp.concatenate([y, jnp.ones((BLK, 1), jnp.float32)], axis=1)
    oh = (lax.broadcasted_iota(jnp.int32, (G, BLK), 0)
          == b_ref[...]).astype(jnp.float32)
    m = _dot(oh, ye)

    @pl.when(i == 0)
    def _():
        acc_ref[...] = m

    @pl.when(i > 0)
    def _():
        acc_ref[...] += m

    @pl.when(i == GRID - 1)
    def _():
        pooled = acc_ref[:, :64] / jnp.maximum(acc_ref[:, 64:65], 1.0)
        out_ref[...] = _dot(pooled, wf_ref[...]) + bf_ref[...]


def _tc_pool(z, batch2d, W, mu, istd, g, be, Wf, bf2d):
    return pl.pallas_call(
        _pool_body,
        grid=(GRID,),
        in_specs=[
            pl.BlockSpec((BLK, 32), lambda i: (i, 0)),
            pl.BlockSpec((1, BLK), lambda i: (0, i)),
            pl.BlockSpec((32, 64), lambda i: (0, 0)),
            pl.BlockSpec((1, 64), lambda i: (0, 0)),
            pl.BlockSpec((1, 64), lambda i: (0, 0)),
            pl.BlockSpec((1, 64), lambda i: (0, 0)),
            pl.BlockSpec((1, 64), lambda i: (0, 0)),
            pl.BlockSpec((64, 10), lambda i: (0, 0)),
            pl.BlockSpec((1, 10), lambda i: (0, 0)),
        ],
        out_specs=pl.BlockSpec((G, 10), lambda i: (0, 0)),
        out_shape=jax.ShapeDtypeStruct((G, 10), jnp.float32),
        scratch_shapes=[pltpu.VMEM((G, 65), jnp.float32)],
    )(z, batch2d, W, mu, istd, g, be, Wf, bf2d)


def kernel(x, edge_index, batch, W1, b1, g1, be1, W2, b2, g2, be2,
           W3, b3, g3, be3, Wf, bf):
    f32 = jnp.float32
    npad = EP - E
    padrow = (N + (jnp.arange(npad, dtype=jnp.int32) % (NP - N)))
    src = jnp.concatenate([edge_index[0].astype(jnp.int32), padrow])
    dst = jnp.concatenate([edge_index[1].astype(jnp.int32), padrow])
    src2d = src.reshape(EP // 128, 128)
    dst2d = dst.reshape(EP // 128, 128)

    x16 = jnp.pad(x.astype(f32), ((0, NP - N), (0, 13)))
    valid = jnp.pad(jnp.ones((N, 1), f32), ((0, NP - N), (0, 0)))
    batch2d = jnp.pad(batch.astype(jnp.int32), (0, NP - N),
                      constant_values=G).reshape(1, NP)

    z16 = jnp.zeros((NP, 16), f32)
    ones128 = jnp.ones((128, 16), f32)

    degp = _sc_deg(dst2d, z16, ones128)
    dinv, t1 = _tc_prep(degp[0], degp[1], valid, x16)

    s1 = _sc_agg(t1, src2d, dst2d, z16)
    z1, M1 = _tc_stats(s1[0], s1[1], t1, dinv)
    W1p = jnp.pad(W1.astype(f32), ((0, 13), (0, 0)))
    mu1, istd1 = _finalize_bn(M1, W1p)
    t2 = _tc_apply1(z1, dinv, W1p, mu1, istd1,
                    g1.reshape(1, -1), be1.reshape(1, -1))

    s2 = _sc_agg(t2, src2d, dst2d, z16)
    z2, M2 = _tc_stats(s2[0], s2[1], t2, dinv)
    mu2, istd2 = _finalize_bn(M2, W2.astype(f32))
    t3a, t3b = _tc_apply2(z2, dinv, W2.astype(f32), mu2, istd2,
                          g2.reshape(1, -1), be2.reshape(1, -1))

    s3a = _sc_agg(t3a, src2d, dst2d, z16)
    s3b = _sc_agg(t3b, src2d, dst2d, z16)
    z3, M3 = _tc_stats2(s3a[0], s3a[1], t3a, s3b[0], s3b[1], t3b, dinv)
    mu3, istd3 = _finalize_bn(M3, W3.astype(f32))

    return _tc_pool(z3, batch2d, W3.astype(f32), mu3, istd3,
                    g3.reshape(1, -1), be3.reshape(1, -1),
                    Wf.astype(f32), bf.reshape(1, -1))
